# baseline (speedup 1.0000x reference)
import functools

import jax
import jax.numpy as jnp
from jax import lax
from jax.experimental import pallas as pl
from jax.experimental.pallas import tpu as pltpu
from jax.experimental.pallas import tpu_sc as plsc

N = 10000
E = 320000
D = 128
NC = 2
NS = 16
NW = NC * NS
EPW = E // NW
CH = 16
NBUF = 9
EPWP = 10080
NCHUNK = EPWP // CH
ROUNDS = NCHUNK // NBUF
NPAD = 10240
SPT = NPAD // NS


def _sc_deg_body(src_hbm, degb_hbm, idx_v, deg_v, tmp2d, degsum, degb_v,
                 sdegs):
    c = lax.axis_index("c")
    s = lax.axis_index("s")
    wid = c * NS + s
    pltpu.sync_copy(src_hbm.at[wid], idx_v)
    zeros16 = jnp.zeros((16,), jnp.float32)

    def zbody(i, _):
        deg_v[pl.ds(i * 16, 16)] = zeros16
        return 0

    lax.fori_loop(0, NPAD // 16, zbody, 0)
    ones16 = jnp.ones((16,), jnp.float32)

    def abody(i, _):
        idx = idx_v[pl.ds(i * 16, 16)]
        plsc.addupdate_scatter(deg_v, [idx], ones16)
        return 0

    lax.fori_loop(0, EPW // 16, abody, 0)
    pltpu.sync_copy(deg_v, sdegs.at[s])
    plsc.subcore_barrier()
    pltpu.sync_copy(sdegs.at[:, pl.ds(s * SPT, SPT)], tmp2d)

    def cbody(j, _):
        acc = tmp2d[0, pl.ds(j * 16, 16)]
        for r in range(1, NS):
            acc = acc + tmp2d[r, pl.ds(j * 16, 16)]
        degsum[pl.ds(j * 16, 16)] = acc
        return 0

    lax.fori_loop(0, SPT // 16, cbody, 0)

    def bbody(j, _):
        v = degsum[pl.ds(j * 16, 16)]
        for i in range(16):
            vec = jnp.full((16,), v[i], jnp.float32)
            for k in range(D // 16):
                degb_v[j * 16 + i, pl.ds(k * 16, 16)] = vec
        return 0

    lax.fori_loop(0, SPT // 16, bbody, 0)
    pltpu.sync_copy(degb_v, degb_hbm.at[c, pl.ds(s * SPT, SPT)])


@functools.cache
def _get_sc_deg():
    mesh = plsc.VectorSubcoreMesh(core_axis_name="c", subcore_axis_name="s",
                                  num_cores=NC, num_subcores=NS)
    return pl.kernel(
        _sc_deg_body,
        out_type=jax.ShapeDtypeStruct((NC, NPAD, D), jnp.float32),
        mesh=mesh,
        compiler_params=pltpu.CompilerParams(needs_layout_passes=False),
        scratch_types=[
            pltpu.VMEM((EPW,), jnp.int32),
            pltpu.VMEM((NPAD,), jnp.float32),
            pltpu.VMEM((NS, SPT), jnp.float32),
            pltpu.VMEM((SPT,), jnp.float32),
            pltpu.VMEM((SPT, D), jnp.float32),
            pltpu.VMEM_SHARED((NS, NPAD), jnp.float32),
        ],
    )


def _sc_prop_body(g_hbm, srcs_hbm, dsts_hbm, part_hbm,
                  src_v, dst_v, rows_v, acc, *sems):
    gsems = sems[:NBUF]
    ssems = sems[NBUF:]
    c = lax.axis_index("c")
    s = lax.axis_index("s")
    wid = c * NS + s
    pltpu.sync_copy(srcs_hbm.at[wid], src_v)
    pltpu.sync_copy(dsts_hbm.at[wid], dst_v)

    zeros16 = jnp.zeros((16,), jnp.float32)

    def zbody(i, _):
        for jj in range(D // 16):
            rows_v[i, pl.ds(jj * 16, 16)] = zeros16
        return 0

    lax.fori_loop(0, NBUF * CH, zbody, 0)
    zr = NBUF * CH
    for q in range(SPT // zr):
        pltpu.sync_copy(rows_v, acc.at[pl.ds(s * SPT + q * zr, zr)])
    rem = SPT % zr
    if rem:
        pltpu.sync_copy(rows_v.at[pl.ds(0, rem)],
                        acc.at[pl.ds(s * SPT + (SPT // zr) * zr, rem)])
    plsc.subcore_barrier()

    for b in range(NBUF):
        sv = src_v[pl.ds(b * CH, CH)]
        pltpu.async_copy(g_hbm.at[sv], rows_v.at[pl.ds(b * CH, CH)], gsems[b])

    def round_body(r, _):
        j0 = r * NBUF
        for b in range(NBUF):
            sv = src_v[pl.ds((j0 + b) * CH, CH)]
            dv = dst_v[pl.ds((j0 + b) * CH, CH)]
            rsl = rows_v.at[pl.ds(b * CH, CH)]
            pltpu.make_async_copy(g_hbm.at[sv], rsl, gsems[b]).wait()
            pltpu.async_copy(rsl, acc.at[dv], ssems[b], add=True)
        for b in range(NBUF):
            dv = dst_v[pl.ds((j0 + b) * CH, CH)]
            rsl = rows_v.at[pl.ds(b * CH, CH)]
            pltpu.make_async_copy(rsl, acc.at[dv], ssems[b]).wait()

            @pl.when(r < ROUNDS - 1)
            def _():
                nsv = src_v[pl.ds((j0 + NBUF + b) * CH, CH)]
                pltpu.async_copy(g_hbm.at[nsv], rsl, gsems[b])
        return 0

    lax.fori_loop(0, ROUNDS, round_body, 0)
    plsc.subcore_barrier()
    sl = pl.ds(s * SPT, SPT)
    pltpu.sync_copy(acc.at[sl], part_hbm.at[c, sl])


@functools.cache
def _get_sc_prop():
    mesh = plsc.VectorSubcoreMesh(core_axis_name="c", subcore_axis_name="s",
                                  num_cores=NC, num_subcores=NS)
    return pl.kernel(
        _sc_prop_body,
        out_type=jax.ShapeDtypeStruct((NC, NPAD, D), jnp.float32),
        mesh=mesh,
        compiler_params=pltpu.CompilerParams(needs_layout_passes=False),
        scratch_types=[
            pltpu.VMEM((EPWP,), jnp.int32),
            pltpu.VMEM((EPWP,), jnp.int32),
            pltpu.VMEM((NBUF * CH, D), jnp.float32),
            pltpu.VMEM_SHARED((NPAD, D), jnp.float32),
        ] + [pltpu.SemaphoreType.DMA] * (2 * NBUF),
    )


BLK = 1000
GRID = N // BLK


def _tc_init_body(degp, x, w, disb_o, g_o, acc_o):
    deg = degp[0] + degp[1]
    disb = jnp.where(deg > 0, 1.0 / jnp.sqrt(jnp.maximum(deg, 1e-12)), 0.0)
    disb_o[...] = disb
    g_o[...] = disb * x[...]
    acc_o[...] = jnp.dot(x[...], w[...], preferred_element_type=jnp.float32)


_tc_init = pl.pallas_call(
    _tc_init_body,
    grid=(GRID,),
    in_specs=[
        pl.BlockSpec((NC, BLK, D), lambda i: (0, i, 0)),
        pl.BlockSpec((BLK, D), lambda i: (i, 0)),
        pl.BlockSpec((D, D), lambda i: (0, 0)),
    ],
    out_specs=[
        pl.BlockSpec((BLK, D), lambda i: (i, 0)),
        pl.BlockSpec((BLK, D), lambda i: (i, 0)),
        pl.BlockSpec((BLK, D), lambda i: (i, 0)),
    ],
    out_shape=[jax.ShapeDtypeStruct((N, D), jnp.float32)] * 3,
)


def _tc_step_body(part, disb, tprev, accin, w, t_o, g_o, acc_o, *, alpha, beta):
    p = part[0] + part[1]
    t = alpha * disb[...] * p + beta * tprev[...]
    t_o[...] = t
    g_o[...] = disb[...] * t
    acc_o[...] = accin[...] + jnp.dot(t, w[...],
                                      preferred_element_type=jnp.float32)


def _make_tc_step(alpha, beta):
    return pl.pallas_call(
        functools.partial(_tc_step_body, alpha=alpha, beta=beta),
        grid=(GRID,),
        in_specs=[
            pl.BlockSpec((NC, BLK, D), lambda i: (0, i, 0)),
            pl.BlockSpec((BLK, D), lambda i: (i, 0)),
            pl.BlockSpec((BLK, D), lambda i: (i, 0)),
            pl.BlockSpec((BLK, D), lambda i: (i, 0)),
            pl.BlockSpec((D, D), lambda i: (0, 0)),
        ],
        out_specs=[
            pl.BlockSpec((BLK, D), lambda i: (i, 0)),
            pl.BlockSpec((BLK, D), lambda i: (i, 0)),
            pl.BlockSpec((BLK, D), lambda i: (i, 0)),
        ],
        out_shape=[jax.ShapeDtypeStruct((N, D), jnp.float32)] * 3,
    )


_tc_step1 = _make_tc_step(-1.0, 0.0)
_tc_step2 = _make_tc_step(-2.0, -1.0)


def _tc_fin_body(part, disb, tprev2, accin, w4, bias, wnext, h_o, g_o, accn_o):
    p = part[0] + part[1]
    t4 = -2.0 * disb[...] * p - tprev2[...]
    o = accin[...] + jnp.dot(t4, w4[...],
                             preferred_element_type=jnp.float32) + bias[...]
    h = jnp.maximum(o, 0.0)
    h_o[...] = h
    g_o[...] = disb[...] * h
    accn_o[...] = jnp.dot(h, wnext[...], preferred_element_type=jnp.float32)


_tc_fin = pl.pallas_call(
    _tc_fin_body,
    grid=(GRID,),
    in_specs=[
        pl.BlockSpec((NC, BLK, D), lambda i: (0, i, 0)),
        pl.BlockSpec((BLK, D), lambda i: (i, 0)),
        pl.BlockSpec((BLK, D), lambda i: (i, 0)),
        pl.BlockSpec((BLK, D), lambda i: (i, 0)),
        pl.BlockSpec((D, D), lambda i: (0, 0)),
        pl.BlockSpec((1, D), lambda i: (0, 0)),
        pl.BlockSpec((D, D), lambda i: (0, 0)),
    ],
    out_specs=[
        pl.BlockSpec((BLK, D), lambda i: (i, 0)),
        pl.BlockSpec((BLK, D), lambda i: (i, 0)),
        pl.BlockSpec((BLK, D), lambda i: (i, 0)),
    ],
    out_shape=[jax.ShapeDtypeStruct((N, D), jnp.float32)] * 3,
)


def _tc_fin_last_body(part, disb, tprev2, accin, w4, bias, out_o):
    p = part[0] + part[1]
    t4 = -2.0 * disb[...] * p - tprev2[...]
    out_o[...] = accin[...] + jnp.dot(
        t4, w4[...], preferred_element_type=jnp.float32) + bias[...]


_tc_fin_last = pl.pallas_call(
    _tc_fin_last_body,
    grid=(GRID,),
    in_specs=[
        pl.BlockSpec((NC, BLK, D), lambda i: (0, i, 0)),
        pl.BlockSpec((BLK, D), lambda i: (i, 0)),
        pl.BlockSpec((BLK, D), lambda i: (i, 0)),
        pl.BlockSpec((BLK, D), lambda i: (i, 0)),
        pl.BlockSpec((D, D), lambda i: (0, 0)),
        pl.BlockSpec((1, D), lambda i: (0, 0)),
    ],
    out_specs=pl.BlockSpec((BLK, D), lambda i: (i, 0)),
    out_shape=jax.ShapeDtypeStruct((N, D), jnp.float32),
)


def kernel(x, edge_index, W, b):
    src = edge_index[0]
    dst = edge_index[1]
    npad_e = EPWP - EPW
    pad_src = (jnp.arange(NW * npad_e, dtype=jnp.int32) * 131 % N
               ).reshape(NW, npad_e)
    srcs = jnp.concatenate([src.reshape(NW, EPW), pad_src], axis=1)
    pad_dst = (N + jnp.arange(NW * npad_e, dtype=jnp.int32) % (NPAD - N)
               ).reshape(NW, npad_e)
    dsts = jnp.concatenate([dst.reshape(NW, EPW), pad_dst], axis=1)

    _sc_deg = _get_sc_deg()
    _sc_prop = _get_sc_prop()
    degp = _sc_deg(src.reshape(NW, EPW))
    disb, g, acc = _tc_init(degp, x, W[0, 0])

    h = x
    out = None
    for l in range(3):
        t0 = h
        part = _sc_prop(g, srcs, dsts)
        t1, g, acc = _tc_step1(part, disb, t0, acc, W[l, 1])
        part = _sc_prop(g, srcs, dsts)
        t2, g, acc = _tc_step2(part, disb, t0, acc, W[l, 2])
        part = _sc_prop(g, srcs, dsts)
        t3, g, acc = _tc_step2(part, disb, t1, acc, W[l, 3])
        part = _sc_prop(g, srcs, dsts)
        if l < 2:
            h, g, acc = _tc_fin(part, disb, t2, acc, W[l, 4],
                                b[l].reshape(1, D), W[l + 1, 0])
        else:
            out = _tc_fin_last(part, disb, t2, acc, W[l, 4],
                               b[l].reshape(1, D))
    return out


# speedup vs baseline: 1.0084x; 1.0084x over previous
import functools

import jax
import jax.numpy as jnp
from jax import lax
from jax.experimental import pallas as pl
from jax.experimental.pallas import tpu as pltpu
from jax.experimental.pallas import tpu_sc as plsc

N = 10000
E = 320000
D = 128
NC = 2
NS = 16
NW = NC * NS
EPW = E // NW
CH = 16
NBUF = 6
EPWP = 10080
NCHUNK = EPWP // CH
ROUNDS = NCHUNK // NBUF
NPAD = 10240
SPT = NPAD // NS


def _sc_deg_body(src_hbm, degb_hbm, idx_v, deg_v, tmp2d, degsum, degb_v,
                 sdegs):
    c = lax.axis_index("c")
    s = lax.axis_index("s")
    wid = c * NS + s
    pltpu.sync_copy(src_hbm.at[wid], idx_v)
    zeros16 = jnp.zeros((16,), jnp.float32)

    def zbody(i, _):
        deg_v[pl.ds(i * 16, 16)] = zeros16
        return 0

    lax.fori_loop(0, NPAD // 16, zbody, 0)
    ones16 = jnp.ones((16,), jnp.float32)

    def abody(i, _):
        idx = idx_v[pl.ds(i * 16, 16)]
        plsc.addupdate_scatter(deg_v, [idx], ones16)
        return 0

    lax.fori_loop(0, EPW // 16, abody, 0)
    pltpu.sync_copy(deg_v, sdegs.at[s])
    plsc.subcore_barrier()
    pltpu.sync_copy(sdegs.at[:, pl.ds(s * SPT, SPT)], tmp2d)

    def cbody(j, _):
        acc = tmp2d[0, pl.ds(j * 16, 16)]
        for r in range(1, NS):
            acc = acc + tmp2d[r, pl.ds(j * 16, 16)]
        degsum[pl.ds(j * 16, 16)] = acc
        return 0

    lax.fori_loop(0, SPT // 16, cbody, 0)

    def bbody(j, _):
        v = degsum[pl.ds(j * 16, 16)]
        for i in range(16):
            vec = jnp.full((16,), v[i], jnp.float32)
            for k in range(D // 16):
                degb_v[j * 16 + i, pl.ds(k * 16, 16)] = vec
        return 0

    lax.fori_loop(0, SPT // 16, bbody, 0)
    pltpu.sync_copy(degb_v, degb_hbm.at[c, pl.ds(s * SPT, SPT)])


@functools.cache
def _get_sc_deg():
    mesh = plsc.VectorSubcoreMesh(core_axis_name="c", subcore_axis_name="s",
                                  num_cores=NC, num_subcores=NS)
    return pl.kernel(
        _sc_deg_body,
        out_type=jax.ShapeDtypeStruct((NC, NPAD, D), jnp.float32),
        mesh=mesh,
        compiler_params=pltpu.CompilerParams(needs_layout_passes=False),
        scratch_types=[
            pltpu.VMEM((EPW,), jnp.int32),
            pltpu.VMEM((NPAD,), jnp.float32),
            pltpu.VMEM((NS, SPT), jnp.float32),
            pltpu.VMEM((SPT,), jnp.float32),
            pltpu.VMEM((SPT, D), jnp.float32),
            pltpu.VMEM_SHARED((NS, NPAD), jnp.float32),
        ],
    )


def _sc_prop_body(g_hbm, srcs_hbm, dsts_hbm, part_hbm,
                  src_v, dst_v, rows_v, acc, *sems):
    gsems = sems[:NBUF]
    ssems = sems[NBUF:]
    c = lax.axis_index("c")
    s = lax.axis_index("s")
    wid = c * NS + s
    pltpu.sync_copy(srcs_hbm.at[wid], src_v)
    pltpu.sync_copy(dsts_hbm.at[wid], dst_v)

    zeros16 = jnp.zeros((16,), jnp.float32)

    def zbody(i, _):
        for jj in range(D // 16):
            rows_v[i, pl.ds(jj * 16, 16)] = zeros16
        return 0

    lax.fori_loop(0, NBUF * CH, zbody, 0)
    zr = NBUF * CH
    for q in range(SPT // zr):
        pltpu.sync_copy(rows_v, acc.at[pl.ds(s * SPT + q * zr, zr)])
    rem = SPT % zr
    if rem:
        pltpu.sync_copy(rows_v.at[pl.ds(0, rem)],
                        acc.at[pl.ds(s * SPT + (SPT // zr) * zr, rem)])
    plsc.subcore_barrier()

    for b in range(NBUF):
        sv = src_v[pl.ds(b * CH, CH)]
        pltpu.async_copy(g_hbm.at[sv], rows_v.at[pl.ds(b * CH, CH)], gsems[b])

    def round_body(r, _):
        j0 = r * NBUF
        for b in range(NBUF):
            sv = src_v[pl.ds((j0 + b) * CH, CH)]
            dv = dst_v[pl.ds((j0 + b) * CH, CH)]
            rsl = rows_v.at[pl.ds(b * CH, CH)]
            pltpu.make_async_copy(g_hbm.at[sv], rsl, gsems[b]).wait()
            pltpu.async_copy(rsl, acc.at[dv], ssems[b], add=True)
        for b in range(NBUF):
            dv = dst_v[pl.ds((j0 + b) * CH, CH)]
            rsl = rows_v.at[pl.ds(b * CH, CH)]
            pltpu.make_async_copy(rsl, acc.at[dv], ssems[b]).wait()

            @pl.when(r < ROUNDS - 1)
            def _():
                nsv = src_v[pl.ds((j0 + NBUF + b) * CH, CH)]
                pltpu.async_copy(g_hbm.at[nsv], rsl, gsems[b])
        return 0

    lax.fori_loop(0, ROUNDS, round_body, 0)
    plsc.subcore_barrier()
    sl = pl.ds(s * SPT, SPT)
    pltpu.sync_copy(acc.at[sl], part_hbm.at[c, sl])


@functools.cache
def _get_sc_prop():
    mesh = plsc.VectorSubcoreMesh(core_axis_name="c", subcore_axis_name="s",
                                  num_cores=NC, num_subcores=NS)
    return pl.kernel(
        _sc_prop_body,
        out_type=jax.ShapeDtypeStruct((NC, NPAD, D), jnp.float32),
        mesh=mesh,
        compiler_params=pltpu.CompilerParams(needs_layout_passes=False),
        scratch_types=[
            pltpu.VMEM((EPWP,), jnp.int32),
            pltpu.VMEM((EPWP,), jnp.int32),
            pltpu.VMEM((NBUF * CH, D), jnp.float32),
            pltpu.VMEM_SHARED((NPAD, D), jnp.float32),
        ] + [pltpu.SemaphoreType.DMA] * (2 * NBUF),
    )


BLK = 1000
GRID = N // BLK


def _tc_init_body(degp, x, w, disb_o, g_o, acc_o):
    deg = degp[0] + degp[1]
    disb = jnp.where(deg > 0, 1.0 / jnp.sqrt(jnp.maximum(deg, 1e-12)), 0.0)
    disb_o[...] = disb
    g_o[...] = disb * x[...]
    acc_o[...] = jnp.dot(x[...], w[...], preferred_element_type=jnp.float32)


_tc_init = pl.pallas_call(
    _tc_init_body,
    grid=(GRID,),
    in_specs=[
        pl.BlockSpec((NC, BLK, D), lambda i: (0, i, 0)),
        pl.BlockSpec((BLK, D), lambda i: (i, 0)),
        pl.BlockSpec((D, D), lambda i: (0, 0)),
    ],
    out_specs=[
        pl.BlockSpec((BLK, D), lambda i: (i, 0)),
        pl.BlockSpec((BLK, D), lambda i: (i, 0)),
        pl.BlockSpec((BLK, D), lambda i: (i, 0)),
    ],
    out_shape=[jax.ShapeDtypeStruct((N, D), jnp.float32)] * 3,
)


def _tc_step_body(part, disb, tprev, accin, w, t_o, g_o, acc_o, *, alpha, beta):
    p = part[0] + part[1]
    t = alpha * disb[...] * p + beta * tprev[...]
    t_o[...] = t
    g_o[...] = disb[...] * t
    acc_o[...] = accin[...] + jnp.dot(t, w[...],
                                      preferred_element_type=jnp.float32)


def _make_tc_step(alpha, beta):
    return pl.pallas_call(
        functools.partial(_tc_step_body, alpha=alpha, beta=beta),
        grid=(GRID,),
        in_specs=[
            pl.BlockSpec((NC, BLK, D), lambda i: (0, i, 0)),
            pl.BlockSpec((BLK, D), lambda i: (i, 0)),
            pl.BlockSpec((BLK, D), lambda i: (i, 0)),
            pl.BlockSpec((BLK, D), lambda i: (i, 0)),
            pl.BlockSpec((D, D), lambda i: (0, 0)),
        ],
        out_specs=[
            pl.BlockSpec((BLK, D), lambda i: (i, 0)),
            pl.BlockSpec((BLK, D), lambda i: (i, 0)),
            pl.BlockSpec((BLK, D), lambda i: (i, 0)),
        ],
        out_shape=[jax.ShapeDtypeStruct((N, D), jnp.float32)] * 3,
    )


_tc_step1 = _make_tc_step(-1.0, 0.0)
_tc_step2 = _make_tc_step(-2.0, -1.0)


def _tc_fin_body(part, disb, tprev2, accin, w4, bias, wnext, h_o, g_o, accn_o):
    p = part[0] + part[1]
    t4 = -2.0 * disb[...] * p - tprev2[...]
    o = accin[...] + jnp.dot(t4, w4[...],
                             preferred_element_type=jnp.float32) + bias[...]
    h = jnp.maximum(o, 0.0)
    h_o[...] = h
    g_o[...] = disb[...] * h
    accn_o[...] = jnp.dot(h, wnext[...], preferred_element_type=jnp.float32)


_tc_fin = pl.pallas_call(
    _tc_fin_body,
    grid=(GRID,),
    in_specs=[
        pl.BlockSpec((NC, BLK, D), lambda i: (0, i, 0)),
        pl.BlockSpec((BLK, D), lambda i: (i, 0)),
        pl.BlockSpec((BLK, D), lambda i: (i, 0)),
        pl.BlockSpec((BLK, D), lambda i: (i, 0)),
        pl.BlockSpec((D, D), lambda i: (0, 0)),
        pl.BlockSpec((1, D), lambda i: (0, 0)),
        pl.BlockSpec((D, D), lambda i: (0, 0)),
    ],
    out_specs=[
        pl.BlockSpec((BLK, D), lambda i: (i, 0)),
        pl.BlockSpec((BLK, D), lambda i: (i, 0)),
        pl.BlockSpec((BLK, D), lambda i: (i, 0)),
    ],
    out_shape=[jax.ShapeDtypeStruct((N, D), jnp.float32)] * 3,
)


def _tc_fin_last_body(part, disb, tprev2, accin, w4, bias, out_o):
    p = part[0] + part[1]
    t4 = -2.0 * disb[...] * p - tprev2[...]
    out_o[...] = accin[...] + jnp.dot(
        t4, w4[...], preferred_element_type=jnp.float32) + bias[...]


_tc_fin_last = pl.pallas_call(
    _tc_fin_last_body,
    grid=(GRID,),
    in_specs=[
        pl.BlockSpec((NC, BLK, D), lambda i: (0, i, 0)),
        pl.BlockSpec((BLK, D), lambda i: (i, 0)),
        pl.BlockSpec((BLK, D), lambda i: (i, 0)),
        pl.BlockSpec((BLK, D), lambda i: (i, 0)),
        pl.BlockSpec((D, D), lambda i: (0, 0)),
        pl.BlockSpec((1, D), lambda i: (0, 0)),
    ],
    out_specs=pl.BlockSpec((BLK, D), lambda i: (i, 0)),
    out_shape=jax.ShapeDtypeStruct((N, D), jnp.float32),
)


def kernel(x, edge_index, W, b):
    src = edge_index[0]
    dst = edge_index[1]
    npad_e = EPWP - EPW
    pad_src = (jnp.arange(NW * npad_e, dtype=jnp.int32) * 131 % N
               ).reshape(NW, npad_e)
    srcs = jnp.concatenate([src.reshape(NW, EPW), pad_src], axis=1)
    pad_dst = (N + jnp.arange(NW * npad_e, dtype=jnp.int32) % (NPAD - N)
               ).reshape(NW, npad_e)
    dsts = jnp.concatenate([dst.reshape(NW, EPW), pad_dst], axis=1)

    _sc_deg = _get_sc_deg()
    _sc_prop = _get_sc_prop()
    degp = _sc_deg(src.reshape(NW, EPW))
    disb, g, acc = _tc_init(degp, x, W[0, 0])

    h = x
    out = None
    for l in range(3):
        t0 = h
        part = _sc_prop(g, srcs, dsts)
        t1, g, acc = _tc_step1(part, disb, t0, acc, W[l, 1])
        part = _sc_prop(g, srcs, dsts)
        t2, g, acc = _tc_step2(part, disb, t0, acc, W[l, 2])
        part = _sc_prop(g, srcs, dsts)
        t3, g, acc = _tc_step2(part, disb, t1, acc, W[l, 3])
        part = _sc_prop(g, srcs, dsts)
        if l < 2:
            h, g, acc = _tc_fin(part, disb, t2, acc, W[l, 4],
                                b[l].reshape(1, D), W[l + 1, 0])
        else:
            out = _tc_fin_last(part, disb, t2, acc, W[l, 4],
                               b[l].reshape(1, D))
    return out


# speedup vs baseline: 1.0995x; 1.0904x over previous
import functools

import jax
import jax.numpy as jnp
from jax import lax
from jax.experimental import pallas as pl
from jax.experimental.pallas import tpu as pltpu
from jax.experimental.pallas import tpu_sc as plsc

N = 10000
E = 320000
D = 128
NC = 2
NS = 16
NW = NC * NS
EPW = E // NW
CH = 16
NBUF = 8
EPWP = 10112
NCHUNK = EPWP // CH
ROUNDS = NCHUNK // NBUF
NPAD = 10240
SPT = NPAD // NS


def _sc_deg_body(src_hbm, degb_hbm, idx_v, deg_v, tmp2d, degsum, degb_v,
                 sdegs):
    c = lax.axis_index("c")
    s = lax.axis_index("s")
    wid = c * NS + s
    pltpu.sync_copy(src_hbm.at[wid], idx_v)
    zeros16 = jnp.zeros((16,), jnp.float32)

    def zbody(i, _):
        deg_v[pl.ds(i * 16, 16)] = zeros16
        return 0

    lax.fori_loop(0, NPAD // 16, zbody, 0)
    ones16 = jnp.ones((16,), jnp.float32)

    def abody(i, _):
        idx = idx_v[pl.ds(i * 16, 16)]
        plsc.addupdate_scatter(deg_v, [idx], ones16)
        return 0

    lax.fori_loop(0, EPW // 16, abody, 0)
    pltpu.sync_copy(deg_v, sdegs.at[s])
    plsc.subcore_barrier()
    pltpu.sync_copy(sdegs.at[:, pl.ds(s * SPT, SPT)], tmp2d)

    def cbody(j, _):
        acc = tmp2d[0, pl.ds(j * 16, 16)]
        for r in range(1, NS):
            acc = acc + tmp2d[r, pl.ds(j * 16, 16)]
        degsum[pl.ds(j * 16, 16)] = acc
        return 0

    lax.fori_loop(0, SPT // 16, cbody, 0)

    def bbody(j, _):
        v = degsum[pl.ds(j * 16, 16)]
        for i in range(16):
            vec = jnp.full((16,), v[i], jnp.float32)
            for k in range(D // 16):
                degb_v[j * 16 + i, pl.ds(k * 16, 16)] = vec
        return 0

    lax.fori_loop(0, SPT // 16, bbody, 0)
    pltpu.sync_copy(degb_v, degb_hbm.at[c, pl.ds(s * SPT, SPT)])


@functools.cache
def _get_sc_deg():
    mesh = plsc.VectorSubcoreMesh(core_axis_name="c", subcore_axis_name="s",
                                  num_cores=NC, num_subcores=NS)
    return pl.kernel(
        _sc_deg_body,
        out_type=jax.ShapeDtypeStruct((NC, NPAD, D), jnp.float32),
        mesh=mesh,
        compiler_params=pltpu.CompilerParams(needs_layout_passes=False),
        scratch_types=[
            pltpu.VMEM((EPW,), jnp.int32),
            pltpu.VMEM((NPAD,), jnp.float32),
            pltpu.VMEM((NS, SPT), jnp.float32),
            pltpu.VMEM((SPT,), jnp.float32),
            pltpu.VMEM((SPT, D), jnp.float32),
            pltpu.VMEM_SHARED((NS, NPAD), jnp.float32),
        ],
    )


def _sc_prop_body(g_hbm, srcs_hbm, dsts_hbm, part_hbm,
                  src_v, dst_v, rows_v, acc, *sems):
    gsems = sems[:NBUF]
    ssems = sems[NBUF:]
    c = lax.axis_index("c")
    s = lax.axis_index("s")
    wid = c * NS + s
    pltpu.sync_copy(srcs_hbm.at[wid], src_v)
    pltpu.sync_copy(dsts_hbm.at[wid], dst_v)

    zeros16 = jnp.zeros((16,), jnp.float32)

    def zbody(i, _):
        for jj in range(D // 16):
            rows_v[i, pl.ds(jj * 16, 16)] = zeros16
        return 0

    lax.fori_loop(0, NBUF * CH, zbody, 0)
    zr = NBUF * CH
    for q in range(SPT // zr):
        pltpu.sync_copy(rows_v, acc.at[pl.ds(s * SPT + q * zr, zr)])
    rem = SPT % zr
    if rem:
        pltpu.sync_copy(rows_v.at[pl.ds(0, rem)],
                        acc.at[pl.ds(s * SPT + (SPT // zr) * zr, rem)])
    plsc.subcore_barrier()

    for b in range(NBUF):
        sv = src_v[pl.ds(b * CH, CH)]
        pltpu.async_copy(g_hbm.at[sv], rows_v.at[pl.ds(b * CH, CH)], gsems[b])

    def round_body(r, _):
        j0 = r * NBUF
        for b in range(NBUF):
            sv = src_v[pl.ds((j0 + b) * CH, CH)]
            dv = dst_v[pl.ds((j0 + b) * CH, CH)]
            rsl = rows_v.at[pl.ds(b * CH, CH)]
            pltpu.make_async_copy(g_hbm.at[sv], rsl, gsems[b]).wait()
            pltpu.async_copy(rsl, acc.at[dv], ssems[b], add=True)
        for b in range(NBUF):
            dv = dst_v[pl.ds((j0 + b) * CH, CH)]
            rsl = rows_v.at[pl.ds(b * CH, CH)]
            pltpu.make_async_copy(rsl, acc.at[dv], ssems[b]).wait()

            @pl.when(r < ROUNDS - 1)
            def _():
                nsv = src_v[pl.ds((j0 + NBUF + b) * CH, CH)]
                pltpu.async_copy(g_hbm.at[nsv], rsl, gsems[b])
        return 0

    lax.fori_loop(0, ROUNDS, round_body, 0)
    plsc.subcore_barrier()
    sl = pl.ds(s * SPT, SPT)
    pltpu.sync_copy(acc.at[sl], part_hbm.at[c, sl])


@functools.cache
def _get_sc_prop():
    mesh = plsc.VectorSubcoreMesh(core_axis_name="c", subcore_axis_name="s",
                                  num_cores=NC, num_subcores=NS)
    return pl.kernel(
        _sc_prop_body,
        out_type=jax.ShapeDtypeStruct((NC, NPAD, D), jnp.float32),
        mesh=mesh,
        compiler_params=pltpu.CompilerParams(needs_layout_passes=False),
        scratch_types=[
            pltpu.VMEM((EPWP,), jnp.int32),
            pltpu.VMEM((EPWP,), jnp.int32),
            pltpu.VMEM((NBUF * CH, D), jnp.float32),
            pltpu.VMEM_SHARED((NPAD, D), jnp.float32),
        ] + [pltpu.SemaphoreType.DMA] * (2 * NBUF),
    )


BLK = 1000
GRID = N // BLK


def _tc_init_body(degp, x, w, disb_o, g_o, acc_o):
    deg = degp[0] + degp[1]
    disb = jnp.where(deg > 0, 1.0 / jnp.sqrt(jnp.maximum(deg, 1e-12)), 0.0)
    disb_o[...] = disb
    g_o[...] = disb * x[...]
    acc_o[...] = jnp.dot(x[...], w[...], preferred_element_type=jnp.float32)


_tc_init = pl.pallas_call(
    _tc_init_body,
    grid=(GRID,),
    in_specs=[
        pl.BlockSpec((NC, BLK, D), lambda i: (0, i, 0)),
        pl.BlockSpec((BLK, D), lambda i: (i, 0)),
        pl.BlockSpec((D, D), lambda i: (0, 0)),
    ],
    out_specs=[
        pl.BlockSpec((BLK, D), lambda i: (i, 0)),
        pl.BlockSpec((BLK, D), lambda i: (i, 0)),
        pl.BlockSpec((BLK, D), lambda i: (i, 0)),
    ],
    out_shape=[jax.ShapeDtypeStruct((N, D), jnp.float32)] * 3,
)


def _tc_step_body(part, disb, tprev, accin, w, t_o, g_o, acc_o, *, alpha, beta):
    p = part[0] + part[1]
    t = alpha * disb[...] * p + beta * tprev[...]
    t_o[...] = t
    g_o[...] = disb[...] * t
    acc_o[...] = accin[...] + jnp.dot(t, w[...],
                                      preferred_element_type=jnp.float32)


def _make_tc_step(alpha, beta):
    return pl.pallas_call(
        functools.partial(_tc_step_body, alpha=alpha, beta=beta),
        grid=(GRID,),
        in_specs=[
            pl.BlockSpec((NC, BLK, D), lambda i: (0, i, 0)),
            pl.BlockSpec((BLK, D), lambda i: (i, 0)),
            pl.BlockSpec((BLK, D), lambda i: (i, 0)),
            pl.BlockSpec((BLK, D), lambda i: (i, 0)),
            pl.BlockSpec((D, D), lambda i: (0, 0)),
        ],
        out_specs=[
            pl.BlockSpec((BLK, D), lambda i: (i, 0)),
            pl.BlockSpec((BLK, D), lambda i: (i, 0)),
            pl.BlockSpec((BLK, D), lambda i: (i, 0)),
        ],
        out_shape=[jax.ShapeDtypeStruct((N, D), jnp.float32)] * 3,
    )


_tc_step1 = _make_tc_step(-1.0, 0.0)
_tc_step2 = _make_tc_step(-2.0, -1.0)


def _tc_fin_body(part, disb, tprev2, accin, w4, bias, wnext, h_o, g_o, accn_o):
    p = part[0] + part[1]
    t4 = -2.0 * disb[...] * p - tprev2[...]
    o = accin[...] + jnp.dot(t4, w4[...],
                             preferred_element_type=jnp.float32) + bias[...]
    h = jnp.maximum(o, 0.0)
    h_o[...] = h
    g_o[...] = disb[...] * h
    accn_o[...] = jnp.dot(h, wnext[...], preferred_element_type=jnp.float32)


_tc_fin = pl.pallas_call(
    _tc_fin_body,
    grid=(GRID,),
    in_specs=[
        pl.BlockSpec((NC, BLK, D), lambda i: (0, i, 0)),
        pl.BlockSpec((BLK, D), lambda i: (i, 0)),
        pl.BlockSpec((BLK, D), lambda i: (i, 0)),
        pl.BlockSpec((BLK, D), lambda i: (i, 0)),
        pl.BlockSpec((D, D), lambda i: (0, 0)),
        pl.BlockSpec((1, D), lambda i: (0, 0)),
        pl.BlockSpec((D, D), lambda i: (0, 0)),
    ],
    out_specs=[
        pl.BlockSpec((BLK, D), lambda i: (i, 0)),
        pl.BlockSpec((BLK, D), lambda i: (i, 0)),
        pl.BlockSpec((BLK, D), lambda i: (i, 0)),
    ],
    out_shape=[jax.ShapeDtypeStruct((N, D), jnp.float32)] * 3,
)


def _tc_fin_last_body(part, disb, tprev2, accin, w4, bias, out_o):
    p = part[0] + part[1]
    t4 = -2.0 * disb[...] * p - tprev2[...]
    out_o[...] = accin[...] + jnp.dot(
        t4, w4[...], preferred_element_type=jnp.float32) + bias[...]


_tc_fin_last = pl.pallas_call(
    _tc_fin_last_body,
    grid=(GRID,),
    in_specs=[
        pl.BlockSpec((NC, BLK, D), lambda i: (0, i, 0)),
        pl.BlockSpec((BLK, D), lambda i: (i, 0)),
        pl.BlockSpec((BLK, D), lambda i: (i, 0)),
        pl.BlockSpec((BLK, D), lambda i: (i, 0)),
        pl.BlockSpec((D, D), lambda i: (0, 0)),
        pl.BlockSpec((1, D), lambda i: (0, 0)),
    ],
    out_specs=pl.BlockSpec((BLK, D), lambda i: (i, 0)),
    out_shape=jax.ShapeDtypeStruct((N, D), jnp.float32),
)


def kernel(x, edge_index, W, b):
    src = edge_index[0]
    dst = edge_index[1]
    npad_e = EPWP - EPW
    pad_src = (jnp.arange(NW * npad_e, dtype=jnp.int32) * 131 % N
               ).reshape(NW, npad_e)
    srcs = jnp.concatenate([src.reshape(NW, EPW), pad_src], axis=1)
    pad_dst = (N + jnp.arange(NW * npad_e, dtype=jnp.int32) % (NPAD - N)
               ).reshape(NW, npad_e)
    dsts = jnp.concatenate([dst.reshape(NW, EPW), pad_dst], axis=1)

    _sc_deg = _get_sc_deg()
    _sc_prop = _get_sc_prop()
    degp = _sc_deg(src.reshape(NW, EPW))
    disb, g, acc = _tc_init(degp, x, W[0, 0])

    h = x
    out = None
    for l in range(3):
        t0 = h
        part = _sc_prop(g, srcs, dsts)
        t1, g, acc = _tc_step1(part, disb, t0, acc, W[l, 1])
        part = _sc_prop(g, srcs, dsts)
        t2, g, acc = _tc_step2(part, disb, t0, acc, W[l, 2])
        part = _sc_prop(g, srcs, dsts)
        t3, g, acc = _tc_step2(part, disb, t1, acc, W[l, 3])
        part = _sc_prop(g, srcs, dsts)
        if l < 2:
            h, g, acc = _tc_fin(part, disb, t2, acc, W[l, 4],
                                b[l].reshape(1, D), W[l + 1, 0])
        else:
            out = _tc_fin_last(part, disb, t2, acc, W[l, 4],
                               b[l].reshape(1, D))
    return out


# speedup vs baseline: 1.1147x; 1.0138x over previous
import functools

import jax
import jax.numpy as jnp
from jax import lax
from jax.experimental import pallas as pl
from jax.experimental.pallas import tpu as pltpu
from jax.experimental.pallas import tpu_sc as plsc

N = 10000
E = 320000
D = 128
NC = 2
NS = 16
NW = NC * NS
EPW = E // NW
CH = 16
NBUF = 8
EPWP = 10112
NCHUNK = EPWP // CH
ROUNDS = NCHUNK // NBUF
NPAD = 10240
SPT = NPAD // NS


def _sc_deg_body(src_hbm, degb_hbm, idx_v, deg_v, tmp2d, degsum, degb_v,
                 sdegs):
    c = lax.axis_index("c")
    s = lax.axis_index("s")
    wid = c * NS + s
    pltpu.sync_copy(src_hbm.at[wid], idx_v)
    zeros16 = jnp.zeros((16,), jnp.float32)

    def zbody(i, _):
        deg_v[pl.ds(i * 16, 16)] = zeros16
        return 0

    lax.fori_loop(0, NPAD // 16, zbody, 0)
    ones16 = jnp.ones((16,), jnp.float32)

    def abody(i, _):
        idx = idx_v[pl.ds(i * 16, 16)]
        plsc.addupdate_scatter(deg_v, [idx], ones16)
        return 0

    lax.fori_loop(0, EPW // 16, abody, 0)
    pltpu.sync_copy(deg_v, sdegs.at[s])
    plsc.subcore_barrier()
    pltpu.sync_copy(sdegs.at[:, pl.ds(s * SPT, SPT)], tmp2d)

    def cbody(j, _):
        acc = tmp2d[0, pl.ds(j * 16, 16)]
        for r in range(1, NS):
            acc = acc + tmp2d[r, pl.ds(j * 16, 16)]
        degsum[pl.ds(j * 16, 16)] = acc
        return 0

    lax.fori_loop(0, SPT // 16, cbody, 0)

    def bbody(j, _):
        v = degsum[pl.ds(j * 16, 16)]
        for i in range(16):
            vec = jnp.full((16,), v[i], jnp.float32)
            for k in range(D // 16):
                degb_v[j * 16 + i, pl.ds(k * 16, 16)] = vec
        return 0

    lax.fori_loop(0, SPT // 16, bbody, 0)
    pltpu.sync_copy(degb_v, degb_hbm.at[c, pl.ds(s * SPT, SPT)])


@functools.cache
def _get_sc_deg():
    mesh = plsc.VectorSubcoreMesh(core_axis_name="c", subcore_axis_name="s",
                                  num_cores=NC, num_subcores=NS)
    return pl.kernel(
        _sc_deg_body,
        out_type=jax.ShapeDtypeStruct((NC, NPAD, D), jnp.float32),
        mesh=mesh,
        compiler_params=pltpu.CompilerParams(needs_layout_passes=False),
        scratch_types=[
            pltpu.VMEM((EPW,), jnp.int32),
            pltpu.VMEM((NPAD,), jnp.float32),
            pltpu.VMEM((NS, SPT), jnp.float32),
            pltpu.VMEM((SPT,), jnp.float32),
            pltpu.VMEM((SPT, D), jnp.float32),
            pltpu.VMEM_SHARED((NS, NPAD), jnp.float32),
        ],
    )


def _sc_prop_body(g_hbm, srcs_hbm, dsts_hbm, part_hbm,
                  src_v, dst_v, rows_v, acc, *sems):
    gsems = sems[:NBUF]
    ssems = sems[NBUF:]
    c = lax.axis_index("c")
    s = lax.axis_index("s")
    wid = c * NS + s
    pltpu.sync_copy(srcs_hbm.at[wid], src_v)
    pltpu.sync_copy(dsts_hbm.at[wid], dst_v)

    zeros16 = jnp.zeros((16,), jnp.float32)

    def zbody(i, _):
        for jj in range(D // 16):
            rows_v[i, pl.ds(jj * 16, 16)] = zeros16
        return 0

    lax.fori_loop(0, NBUF * CH, zbody, 0)
    zr = NBUF * CH
    for q in range(SPT // zr):
        pltpu.sync_copy(rows_v, acc.at[pl.ds(s * SPT + q * zr, zr)])
    rem = SPT % zr
    if rem:
        pltpu.sync_copy(rows_v.at[pl.ds(0, rem)],
                        acc.at[pl.ds(s * SPT + (SPT // zr) * zr, rem)])
    plsc.subcore_barrier()

    for b in range(NBUF):
        sv = src_v[pl.ds(b * CH, CH)]
        pltpu.async_copy(g_hbm.at[sv], rows_v.at[pl.ds(b * CH, CH)], gsems[b])

    def round_body(r, _):
        j0 = r * NBUF
        for b in range(NBUF):
            sv = src_v[pl.ds((j0 + b) * CH, CH)]
            dv = dst_v[pl.ds((j0 + b) * CH, CH)]
            rsl = rows_v.at[pl.ds(b * CH, CH)]
            pltpu.make_async_copy(g_hbm.at[sv], rsl, gsems[b]).wait()
            pltpu.async_copy(rsl, acc.at[dv], ssems[b], add=True)
        for b in range(NBUF):
            dv = dst_v[pl.ds((j0 + b) * CH, CH)]
            rsl = rows_v.at[pl.ds(b * CH, CH)]
            pltpu.make_async_copy(rsl, acc.at[dv], ssems[b]).wait()

            @pl.when(r < ROUNDS - 1)
            def _():
                nsv = src_v[pl.ds((j0 + NBUF + b) * CH, CH)]
                pltpu.async_copy(g_hbm.at[nsv], rsl, gsems[b])
        return 0

    lax.fori_loop(0, ROUNDS, round_body, 0)
    plsc.subcore_barrier()
    sl = pl.ds(s * SPT, SPT)
    pltpu.sync_copy(acc.at[sl], part_hbm.at[c, sl])


@functools.cache
def _get_sc_prop():
    mesh = plsc.VectorSubcoreMesh(core_axis_name="c", subcore_axis_name="s",
                                  num_cores=NC, num_subcores=NS)
    return pl.kernel(
        _sc_prop_body,
        out_type=jax.ShapeDtypeStruct((NC, NPAD, D), jnp.float32),
        mesh=mesh,
        compiler_params=pltpu.CompilerParams(needs_layout_passes=False),
        scratch_types=[
            pltpu.VMEM((EPWP,), jnp.int32),
            pltpu.VMEM((EPWP,), jnp.int32),
            pltpu.VMEM((NBUF * CH, D), jnp.float32),
            pltpu.VMEM_SHARED((NPAD, D), jnp.float32),
        ] + [pltpu.SemaphoreType.DMA] * (2 * NBUF),
    )


BLK = 1000
GRID = N // BLK


def _tc_init_body(degp, x, disb_o, g_o):
    deg = degp[0] + degp[1]
    disb = jnp.where(deg > 0, 1.0 / jnp.sqrt(jnp.maximum(deg, 1e-12)), 0.0)
    disb_o[...] = disb
    g_o[...] = disb * x[...]


_tc_init = pl.pallas_call(
    _tc_init_body,
    grid=(GRID,),
    in_specs=[
        pl.BlockSpec((NC, BLK, D), lambda i: (0, i, 0)),
        pl.BlockSpec((BLK, D), lambda i: (i, 0)),
    ],
    out_specs=[
        pl.BlockSpec((BLK, D), lambda i: (i, 0)),
        pl.BlockSpec((BLK, D), lambda i: (i, 0)),
    ],
    out_shape=[jax.ShapeDtypeStruct((N, D), jnp.float32)] * 2,
)


def _tc_tg_body(part, disb, tprev, t_o, g_o, *, alpha, beta):
    p = part[0] + part[1]
    t = alpha * disb[...] * p + beta * tprev[...]
    t_o[...] = t
    g_o[...] = disb[...] * t


def _make_tc_tg(alpha, beta):
    return pl.pallas_call(
        functools.partial(_tc_tg_body, alpha=alpha, beta=beta),
        grid=(GRID,),
        in_specs=[
            pl.BlockSpec((NC, BLK, D), lambda i: (0, i, 0)),
            pl.BlockSpec((BLK, D), lambda i: (i, 0)),
            pl.BlockSpec((BLK, D), lambda i: (i, 0)),
        ],
        out_specs=[
            pl.BlockSpec((BLK, D), lambda i: (i, 0)),
            pl.BlockSpec((BLK, D), lambda i: (i, 0)),
        ],
        out_shape=[jax.ShapeDtypeStruct((N, D), jnp.float32)] * 2,
    )


_tc_tg1 = _make_tc_tg(-1.0, 0.0)
_tc_tg2 = _make_tc_tg(-2.0, -1.0)


def _tc_mm_init_body(t, w, acc_o):
    acc_o[...] = jnp.dot(t[...], w[...], preferred_element_type=jnp.float32)


_tc_mm_init = pl.pallas_call(
    _tc_mm_init_body,
    grid=(GRID,),
    in_specs=[
        pl.BlockSpec((BLK, D), lambda i: (i, 0)),
        pl.BlockSpec((D, D), lambda i: (0, 0)),
    ],
    out_specs=pl.BlockSpec((BLK, D), lambda i: (i, 0)),
    out_shape=jax.ShapeDtypeStruct((N, D), jnp.float32),
)


def _tc_mm_acc_body(t, accin, w, acc_o):
    acc_o[...] = accin[...] + jnp.dot(t[...], w[...],
                                      preferred_element_type=jnp.float32)


_tc_mm_acc = pl.pallas_call(
    _tc_mm_acc_body,
    grid=(GRID,),
    in_specs=[
        pl.BlockSpec((BLK, D), lambda i: (i, 0)),
        pl.BlockSpec((BLK, D), lambda i: (i, 0)),
        pl.BlockSpec((D, D), lambda i: (0, 0)),
    ],
    out_specs=pl.BlockSpec((BLK, D), lambda i: (i, 0)),
    out_shape=jax.ShapeDtypeStruct((N, D), jnp.float32),
)


def _tc_fin_body(part, disb, tprev2, accin, w4, bias, h_o, g_o):
    p = part[0] + part[1]
    t4 = -2.0 * disb[...] * p - tprev2[...]
    o = accin[...] + jnp.dot(t4, w4[...],
                             preferred_element_type=jnp.float32) + bias[...]
    h = jnp.maximum(o, 0.0)
    h_o[...] = h
    g_o[...] = disb[...] * h


_tc_fin = pl.pallas_call(
    _tc_fin_body,
    grid=(GRID,),
    in_specs=[
        pl.BlockSpec((NC, BLK, D), lambda i: (0, i, 0)),
        pl.BlockSpec((BLK, D), lambda i: (i, 0)),
        pl.BlockSpec((BLK, D), lambda i: (i, 0)),
        pl.BlockSpec((BLK, D), lambda i: (i, 0)),
        pl.BlockSpec((D, D), lambda i: (0, 0)),
        pl.BlockSpec((1, D), lambda i: (0, 0)),
    ],
    out_specs=[
        pl.BlockSpec((BLK, D), lambda i: (i, 0)),
        pl.BlockSpec((BLK, D), lambda i: (i, 0)),
    ],
    out_shape=[jax.ShapeDtypeStruct((N, D), jnp.float32)] * 2,
)


def _tc_fin_last_body(part, disb, tprev2, accin, w4, bias, out_o):
    p = part[0] + part[1]
    t4 = -2.0 * disb[...] * p - tprev2[...]
    out_o[...] = accin[...] + jnp.dot(
        t4, w4[...], preferred_element_type=jnp.float32) + bias[...]


_tc_fin_last = pl.pallas_call(
    _tc_fin_last_body,
    grid=(GRID,),
    in_specs=[
        pl.BlockSpec((NC, BLK, D), lambda i: (0, i, 0)),
        pl.BlockSpec((BLK, D), lambda i: (i, 0)),
        pl.BlockSpec((BLK, D), lambda i: (i, 0)),
        pl.BlockSpec((BLK, D), lambda i: (i, 0)),
        pl.BlockSpec((D, D), lambda i: (0, 0)),
        pl.BlockSpec((1, D), lambda i: (0, 0)),
    ],
    out_specs=pl.BlockSpec((BLK, D), lambda i: (i, 0)),
    out_shape=jax.ShapeDtypeStruct((N, D), jnp.float32),
)


def kernel(x, edge_index, W, b):
    src = edge_index[0]
    dst = edge_index[1]
    npad_e = EPWP - EPW
    pad_src = (jnp.arange(NW * npad_e, dtype=jnp.int32) * 131 % N
               ).reshape(NW, npad_e)
    srcs = jnp.concatenate([src.reshape(NW, EPW), pad_src], axis=1)
    pad_dst = (N + jnp.arange(NW * npad_e, dtype=jnp.int32) % (NPAD - N)
               ).reshape(NW, npad_e)
    dsts = jnp.concatenate([dst.reshape(NW, EPW), pad_dst], axis=1)

    _sc_deg = _get_sc_deg()
    _sc_prop = _get_sc_prop()
    degp = _sc_deg(src.reshape(NW, EPW))
    disb, g = _tc_init(degp, x)

    h = x
    out = None
    for l in range(3):
        t0 = h
        acc = _tc_mm_init(t0, W[l, 0])
        part = _sc_prop(g, srcs, dsts)
        t1, g = _tc_tg1(part, disb, t0)
        acc = _tc_mm_acc(t1, acc, W[l, 1])
        part = _sc_prop(g, srcs, dsts)
        t2, g = _tc_tg2(part, disb, t0)
        acc = _tc_mm_acc(t2, acc, W[l, 2])
        part = _sc_prop(g, srcs, dsts)
        t3, g = _tc_tg2(part, disb, t1)
        acc = _tc_mm_acc(t3, acc, W[l, 3])
        part = _sc_prop(g, srcs, dsts)
        if l < 2:
            h, g = _tc_fin(part, disb, t2, acc, W[l, 4], b[l].reshape(1, D))
        else:
            out = _tc_fin_last(part, disb, t2, acc, W[l, 4],
                               b[l].reshape(1, D))
    return out


# speedup vs baseline: 1.1343x; 1.0176x over previous
import functools

import jax
import jax.numpy as jnp
from jax import lax
from jax.experimental import pallas as pl
from jax.experimental.pallas import tpu as pltpu
from jax.experimental.pallas import tpu_sc as plsc

N = 10000
E = 320000
D = 128
NC = 2
NS = 16
NW = NC * NS
EPW = E // NW
CH = 16
NBUF = 8
EPWP = 10112
NCHUNK = EPWP // CH
ROUNDS = NCHUNK // NBUF
NPAD = 10240
SPT = NPAD // NS


def _sc_deg_body(src_hbm, degb_hbm, idx_v, deg_v, tmp2d, degsum, degb_v,
                 sdegs):
    c = lax.axis_index("c")
    s = lax.axis_index("s")
    wid = c * NS + s
    pltpu.sync_copy(src_hbm.at[wid], idx_v)
    zeros16 = jnp.zeros((16,), jnp.float32)

    def zbody(i, _):
        deg_v[pl.ds(i * 16, 16)] = zeros16
        return 0

    lax.fori_loop(0, NPAD // 16, zbody, 0)
    ones16 = jnp.ones((16,), jnp.float32)

    def abody(i, _):
        idx = idx_v[pl.ds(i * 16, 16)]
        plsc.addupdate_scatter(deg_v, [idx], ones16)
        return 0

    lax.fori_loop(0, EPW // 16, abody, 0)
    pltpu.sync_copy(deg_v, sdegs.at[s])
    plsc.subcore_barrier()
    pltpu.sync_copy(sdegs.at[:, pl.ds(s * SPT, SPT)], tmp2d)

    def cbody(j, _):
        acc = tmp2d[0, pl.ds(j * 16, 16)]
        for r in range(1, NS):
            acc = acc + tmp2d[r, pl.ds(j * 16, 16)]
        degsum[pl.ds(j * 16, 16)] = acc
        return 0

    lax.fori_loop(0, SPT // 16, cbody, 0)

    def bbody(j, _):
        v = degsum[pl.ds(j * 16, 16)]
        for i in range(16):
            vec = jnp.full((16,), v[i], jnp.float32)
            for k in range(D // 16):
                degb_v[j * 16 + i, pl.ds(k * 16, 16)] = vec
        return 0

    lax.fori_loop(0, SPT // 16, bbody, 0)
    pltpu.sync_copy(degb_v, degb_hbm.at[c, pl.ds(s * SPT, SPT)])


@functools.cache
def _get_sc_deg():
    mesh = plsc.VectorSubcoreMesh(core_axis_name="c", subcore_axis_name="s",
                                  num_cores=NC, num_subcores=NS)
    return pl.kernel(
        _sc_deg_body,
        out_type=jax.ShapeDtypeStruct((NC, NPAD, D), jnp.float32),
        mesh=mesh,
        compiler_params=pltpu.CompilerParams(needs_layout_passes=False),
        scratch_types=[
            pltpu.VMEM((EPW,), jnp.int32),
            pltpu.VMEM((NPAD,), jnp.float32),
            pltpu.VMEM((NS, SPT), jnp.float32),
            pltpu.VMEM((SPT,), jnp.float32),
            pltpu.VMEM((SPT, D), jnp.float32),
            pltpu.VMEM_SHARED((NS, NPAD), jnp.float32),
        ],
    )


def _sc_prop_body(g_hbm, srcs_hbm, dsts_hbm, part_hbm,
                  src_v, dst_v, rows_v, acc, *sems):
    gsems = sems[:NBUF]
    ssems = sems[NBUF:]
    c = lax.axis_index("c")
    s = lax.axis_index("s")
    wid = c * NS + s
    pltpu.async_copy(srcs_hbm.at[wid], src_v, gsems[0])
    pltpu.async_copy(dsts_hbm.at[wid], dst_v, gsems[1])

    zeros16 = jnp.zeros((16,), jnp.float32)

    def zbody(i, _):
        for jj in range(D // 16):
            rows_v[i, pl.ds(jj * 16, 16)] = zeros16
        return 0

    lax.fori_loop(0, NBUF * CH, zbody, 0)
    zr = NBUF * CH
    nz = SPT // zr
    for q in range(nz):
        pltpu.async_copy(rows_v, acc.at[pl.ds(s * SPT + q * zr, zr)],
                         ssems[q % NBUF])
    rem = SPT % zr
    if rem:
        pltpu.async_copy(rows_v.at[pl.ds(0, rem)],
                         acc.at[pl.ds(s * SPT + nz * zr, rem)],
                         ssems[nz % NBUF])
    pltpu.make_async_copy(srcs_hbm.at[wid], src_v, gsems[0]).wait()
    pltpu.make_async_copy(dsts_hbm.at[wid], dst_v, gsems[1]).wait()
    for q in range(nz):
        pltpu.make_async_copy(
            rows_v, acc.at[pl.ds(s * SPT + q * zr, zr)], ssems[q % NBUF]
        ).wait()
    if rem:
        pltpu.make_async_copy(
            rows_v.at[pl.ds(0, rem)],
            acc.at[pl.ds(s * SPT + nz * zr, rem)], ssems[nz % NBUF]).wait()
    plsc.subcore_barrier()

    for b in range(NBUF):
        sv = src_v[pl.ds(b * CH, CH)]
        pltpu.async_copy(g_hbm.at[sv], rows_v.at[pl.ds(b * CH, CH)], gsems[b])

    def round_body(r, _):
        j0 = r * NBUF
        for b in range(NBUF):
            sv = src_v[pl.ds((j0 + b) * CH, CH)]
            dv = dst_v[pl.ds((j0 + b) * CH, CH)]
            rsl = rows_v.at[pl.ds(b * CH, CH)]
            pltpu.make_async_copy(g_hbm.at[sv], rsl, gsems[b]).wait()
            pltpu.async_copy(rsl, acc.at[dv], ssems[b], add=True)
        for b in range(NBUF):
            dv = dst_v[pl.ds((j0 + b) * CH, CH)]
            rsl = rows_v.at[pl.ds(b * CH, CH)]
            pltpu.make_async_copy(rsl, acc.at[dv], ssems[b]).wait()

            @pl.when(r < ROUNDS - 1)
            def _():
                nsv = src_v[pl.ds((j0 + NBUF + b) * CH, CH)]
                pltpu.async_copy(g_hbm.at[nsv], rsl, gsems[b])
        return 0

    lax.fori_loop(0, ROUNDS, round_body, 0)
    plsc.subcore_barrier()
    sl = pl.ds(s * SPT, SPT)
    pltpu.sync_copy(acc.at[sl], part_hbm.at[c, sl])


@functools.cache
def _get_sc_prop():
    mesh = plsc.VectorSubcoreMesh(core_axis_name="c", subcore_axis_name="s",
                                  num_cores=NC, num_subcores=NS)
    return pl.kernel(
        _sc_prop_body,
        out_type=jax.ShapeDtypeStruct((NC, NPAD, D), jnp.float32),
        mesh=mesh,
        compiler_params=pltpu.CompilerParams(needs_layout_passes=False),
        scratch_types=[
            pltpu.VMEM((EPWP,), jnp.int32),
            pltpu.VMEM((EPWP,), jnp.int32),
            pltpu.VMEM((NBUF * CH, D), jnp.float32),
            pltpu.VMEM_SHARED((NPAD, D), jnp.float32),
        ] + [pltpu.SemaphoreType.DMA] * (2 * NBUF),
    )


BLK = 1000
GRID = N // BLK


def _tc_init_body(degp, x, disb_o, g_o):
    deg = degp[0] + degp[1]
    disb = jnp.where(deg > 0, 1.0 / jnp.sqrt(jnp.maximum(deg, 1e-12)), 0.0)
    disb_o[...] = disb
    g_o[...] = disb * x[...]


_tc_init = pl.pallas_call(
    _tc_init_body,
    grid=(GRID,),
    in_specs=[
        pl.BlockSpec((NC, BLK, D), lambda i: (0, i, 0)),
        pl.BlockSpec((BLK, D), lambda i: (i, 0)),
    ],
    out_specs=[
        pl.BlockSpec((BLK, D), lambda i: (i, 0)),
        pl.BlockSpec((BLK, D), lambda i: (i, 0)),
    ],
    out_shape=[jax.ShapeDtypeStruct((N, D), jnp.float32)] * 2,
)


def _tc_tg_body(part, disb, tprev, t_o, g_o, *, alpha, beta):
    p = part[0] + part[1]
    t = alpha * disb[...] * p + beta * tprev[...]
    t_o[...] = t
    g_o[...] = disb[...] * t


def _make_tc_tg(alpha, beta):
    return pl.pallas_call(
        functools.partial(_tc_tg_body, alpha=alpha, beta=beta),
        grid=(GRID,),
        in_specs=[
            pl.BlockSpec((NC, BLK, D), lambda i: (0, i, 0)),
            pl.BlockSpec((BLK, D), lambda i: (i, 0)),
            pl.BlockSpec((BLK, D), lambda i: (i, 0)),
        ],
        out_specs=[
            pl.BlockSpec((BLK, D), lambda i: (i, 0)),
            pl.BlockSpec((BLK, D), lambda i: (i, 0)),
        ],
        out_shape=[jax.ShapeDtypeStruct((N, D), jnp.float32)] * 2,
    )


def _tc_tg1_body(part, disb, t_o, g_o):
    p = part[0] + part[1]
    t = -disb[...] * p
    t_o[...] = t
    g_o[...] = disb[...] * t


_tc_tg1 = pl.pallas_call(
    _tc_tg1_body,
    grid=(GRID,),
    in_specs=[
        pl.BlockSpec((NC, BLK, D), lambda i: (0, i, 0)),
        pl.BlockSpec((BLK, D), lambda i: (i, 0)),
    ],
    out_specs=[
        pl.BlockSpec((BLK, D), lambda i: (i, 0)),
        pl.BlockSpec((BLK, D), lambda i: (i, 0)),
    ],
    out_shape=[jax.ShapeDtypeStruct((N, D), jnp.float32)] * 2,
)

_tc_tg2 = _make_tc_tg(-2.0, -1.0)


def _tc_mm_init_body(t, w, acc_o):
    acc_o[...] = jnp.dot(t[...], w[...], preferred_element_type=jnp.float32)


_tc_mm_init = pl.pallas_call(
    _tc_mm_init_body,
    grid=(GRID,),
    in_specs=[
        pl.BlockSpec((BLK, D), lambda i: (i, 0)),
        pl.BlockSpec((D, D), lambda i: (0, 0)),
    ],
    out_specs=pl.BlockSpec((BLK, D), lambda i: (i, 0)),
    out_shape=jax.ShapeDtypeStruct((N, D), jnp.float32),
)


def _tc_mm_acc_body(t, accin, w, acc_o):
    acc_o[...] = accin[...] + jnp.dot(t[...], w[...],
                                      preferred_element_type=jnp.float32)


_tc_mm_acc = pl.pallas_call(
    _tc_mm_acc_body,
    grid=(GRID,),
    in_specs=[
        pl.BlockSpec((BLK, D), lambda i: (i, 0)),
        pl.BlockSpec((BLK, D), lambda i: (i, 0)),
        pl.BlockSpec((D, D), lambda i: (0, 0)),
    ],
    out_specs=pl.BlockSpec((BLK, D), lambda i: (i, 0)),
    out_shape=jax.ShapeDtypeStruct((N, D), jnp.float32),
)


def _tc_fin_body(part, disb, tprev2, accin, w4, bias, h_o, g_o):
    p = part[0] + part[1]
    t4 = -2.0 * disb[...] * p - tprev2[...]
    o = accin[...] + jnp.dot(t4, w4[...],
                             preferred_element_type=jnp.float32) + bias[...]
    h = jnp.maximum(o, 0.0)
    h_o[...] = h
    g_o[...] = disb[...] * h


_tc_fin = pl.pallas_call(
    _tc_fin_body,
    grid=(GRID,),
    in_specs=[
        pl.BlockSpec((NC, BLK, D), lambda i: (0, i, 0)),
        pl.BlockSpec((BLK, D), lambda i: (i, 0)),
        pl.BlockSpec((BLK, D), lambda i: (i, 0)),
        pl.BlockSpec((BLK, D), lambda i: (i, 0)),
        pl.BlockSpec((D, D), lambda i: (0, 0)),
        pl.BlockSpec((1, D), lambda i: (0, 0)),
    ],
    out_specs=[
        pl.BlockSpec((BLK, D), lambda i: (i, 0)),
        pl.BlockSpec((BLK, D), lambda i: (i, 0)),
    ],
    out_shape=[jax.ShapeDtypeStruct((N, D), jnp.float32)] * 2,
)


def _tc_fin_last_body(part, disb, tprev2, accin, w4, bias, out_o):
    p = part[0] + part[1]
    t4 = -2.0 * disb[...] * p - tprev2[...]
    out_o[...] = accin[...] + jnp.dot(
        t4, w4[...], preferred_element_type=jnp.float32) + bias[...]


_tc_fin_last = pl.pallas_call(
    _tc_fin_last_body,
    grid=(GRID,),
    in_specs=[
        pl.BlockSpec((NC, BLK, D), lambda i: (0, i, 0)),
        pl.BlockSpec((BLK, D), lambda i: (i, 0)),
        pl.BlockSpec((BLK, D), lambda i: (i, 0)),
        pl.BlockSpec((BLK, D), lambda i: (i, 0)),
        pl.BlockSpec((D, D), lambda i: (0, 0)),
        pl.BlockSpec((1, D), lambda i: (0, 0)),
    ],
    out_specs=pl.BlockSpec((BLK, D), lambda i: (i, 0)),
    out_shape=jax.ShapeDtypeStruct((N, D), jnp.float32),
)


def kernel(x, edge_index, W, b):
    src = edge_index[0]
    dst = edge_index[1]
    npad_e = EPWP - EPW
    pad_src = (jnp.arange(NW * npad_e, dtype=jnp.int32) * 131 % N
               ).reshape(NW, npad_e)
    srcs = jnp.concatenate([src.reshape(NW, EPW), pad_src], axis=1)
    pad_dst = (N + jnp.arange(NW * npad_e, dtype=jnp.int32) % (NPAD - N)
               ).reshape(NW, npad_e)
    dsts = jnp.concatenate([dst.reshape(NW, EPW), pad_dst], axis=1)

    _sc_deg = _get_sc_deg()
    _sc_prop = _get_sc_prop()
    degp = _sc_deg(src.reshape(NW, EPW))
    disb, g = _tc_init(degp, x)

    h = x
    out = None
    for l in range(3):
        t0 = h
        acc = _tc_mm_init(t0, W[l, 0])
        part = _sc_prop(g, srcs, dsts)
        t1, g = _tc_tg1(part, disb)
        acc = _tc_mm_acc(t1, acc, W[l, 1])
        part = _sc_prop(g, srcs, dsts)
        t2, g = _tc_tg2(part, disb, t0)
        acc = _tc_mm_acc(t2, acc, W[l, 2])
        part = _sc_prop(g, srcs, dsts)
        t3, g = _tc_tg2(part, disb, t1)
        acc = _tc_mm_acc(t3, acc, W[l, 3])
        part = _sc_prop(g, srcs, dsts)
        if l < 2:
            h, g = _tc_fin(part, disb, t2, acc, W[l, 4], b[l].reshape(1, D))
        else:
            out = _tc_fin_last(part, disb, t2, acc, W[l, 4],
                               b[l].reshape(1, D))
    return out
